# Initial kernel scaffold; baseline (speedup 1.0000x reference)
import jax
import jax.numpy as jnp
from jax.experimental import pallas as pl


def kernel(x, edge_index, edge_weight, edge_attr, lin1_w, lin2_w, lin2_b, mlp_w1, mlp_b1, mlp_w2, mlp_b2, lin_w, lin_b):
    raise NotImplementedError("write your pallas kernel here")


# speedup vs baseline: 1.6368x; 1.6368x over previous
import functools
import math

import jax
import jax.numpy as jnp
from jax import lax
from jax.experimental import pallas as pl
from jax.experimental.pallas import tpu as pltpu
from jax.experimental.pallas import tpu_sc as plsc

N_NODES = 10000
HIDDEN = 128
NUM_FILTERS = 128
NUM_GAUSSIANS = 50
N_EDGES = 320000
CUTOFF = 10.0
SHIFT = float(math.log(2.0))

NC, NS = 2, 16
NW = NC * NS
EPW = N_EDGES // NW
CH = 80
NCH = EPW // CH
SLAB_OFF = 624
SLAB = 640


def _ssp(v):
    return jnp.maximum(v, 0.0) + jnp.log1p(jnp.exp(-jnp.abs(v))) - SHIFT


def _xf_body(x_ref, w_ref, o_ref):
    o_ref[...] = jnp.dot(x_ref[...], w_ref[...],
                         preferred_element_type=jnp.float32)


def _compute_xf(x, lin1_w):
    BR = 2000
    return pl.pallas_call(
        _xf_body,
        grid=(N_NODES // BR,),
        in_specs=[pl.BlockSpec((BR, HIDDEN), lambda i: (i, 0)),
                  pl.BlockSpec((HIDDEN, NUM_FILTERS), lambda i: (0, 0))],
        out_specs=pl.BlockSpec((BR, NUM_FILTERS), lambda i: (i, 0)),
        out_shape=jax.ShapeDtypeStruct((N_NODES, NUM_FILTERS), jnp.float32),
    )(x, lin1_w)


def _msg_body(ea_ref, ew_ref, g_ref, w1_ref, b1_ref, w2_ref, b2_ref, o_ref):
    h = jnp.dot(ea_ref[...], w1_ref[...],
                preferred_element_type=jnp.float32) + b1_ref[...]
    h = _ssp(h)
    w = jnp.dot(h, w2_ref[...],
                preferred_element_type=jnp.float32) + b2_ref[...]
    cfac = 0.5 * (jnp.cos(ew_ref[...] * (math.pi / CUTOFF)) + 1.0)
    o_ref[...] = w * cfac * g_ref[...]


def _compute_msg(edge_attr, ew2, g, w1, b1, w2, b2):
    BE = 2000
    return pl.pallas_call(
        _msg_body,
        grid=(N_EDGES // BE,),
        in_specs=[pl.BlockSpec((BE, NUM_GAUSSIANS), lambda i: (i, 0)),
                  pl.BlockSpec((BE, 1), lambda i: (i, 0)),
                  pl.BlockSpec((BE, NUM_FILTERS), lambda i: (i, 0)),
                  pl.BlockSpec((NUM_GAUSSIANS, NUM_FILTERS), lambda i: (0, 0)),
                  pl.BlockSpec((1, NUM_FILTERS), lambda i: (0, 0)),
                  pl.BlockSpec((NUM_FILTERS, NUM_FILTERS), lambda i: (0, 0)),
                  pl.BlockSpec((1, NUM_FILTERS), lambda i: (0, 0))],
        out_specs=pl.BlockSpec((BE, NUM_FILTERS), lambda i: (i, 0)),
        out_shape=jax.ShapeDtypeStruct((N_EDGES, NUM_FILTERS), jnp.float32),
    )(edge_attr, ew2, g, w1, b1, w2, b2)


def _tail_body(p_ref, w2_ref, b2_ref, lw_ref, lb_ref, o_ref):
    agg = p_ref[0] + p_ref[1]
    t = jnp.dot(agg, w2_ref[...],
                preferred_element_type=jnp.float32) + b2_ref[...]
    t = _ssp(t)
    o_ref[...] = jnp.dot(t, lw_ref[...],
                         preferred_element_type=jnp.float32) + lb_ref[...]


def _compute_tail(parts, lin2_w, lin2_b, lin_w, lin_b):
    BR = 2000
    return pl.pallas_call(
        _tail_body,
        grid=(N_NODES // BR,),
        in_specs=[pl.BlockSpec((2, BR, NUM_FILTERS), lambda i: (0, i, 0)),
                  pl.BlockSpec((NUM_FILTERS, HIDDEN), lambda i: (0, 0)),
                  pl.BlockSpec((1, HIDDEN), lambda i: (0, 0)),
                  pl.BlockSpec((HIDDEN, HIDDEN), lambda i: (0, 0)),
                  pl.BlockSpec((1, HIDDEN), lambda i: (0, 0))],
        out_specs=pl.BlockSpec((BR, HIDDEN), lambda i: (i, 0)),
        out_shape=jax.ShapeDtypeStruct((N_NODES, HIDDEN), jnp.float32),
    )(parts, lin2_w, lin2_b, lin_w, lin_b)


_sc_mesh = plsc.VectorSubcoreMesh(core_axis_name="c", subcore_axis_name="s")


@functools.partial(
    pl.kernel, mesh=_sc_mesh,
    out_type=jax.ShapeDtypeStruct((N_EDGES, NUM_FILTERS), jnp.float32),
    scratch_types=[pltpu.VMEM((CH,), jnp.int32),
                   pltpu.VMEM((CH, NUM_FILTERS), jnp.float32),
                   pltpu.SemaphoreType.DMA])
def _sc_gather(xf_hbm, col_hbm, out_hbm, idx_v, rows_v, sem):
    c = lax.axis_index("c")
    s = lax.axis_index("s")
    wid = s * NC + c
    base = wid * EPW

    def body(i, carry):
        off = base + i * CH
        pltpu.sync_copy(col_hbm.at[pl.ds(off, CH)], idx_v)
        pltpu.async_copy(xf_hbm.at[idx_v], rows_v, sem).wait()
        pltpu.sync_copy(rows_v, out_hbm.at[pl.ds(off, CH)])
        return carry

    lax.fori_loop(0, NCH, body, 0)


@functools.partial(
    pl.kernel, mesh=_sc_mesh,
    out_type=jax.ShapeDtypeStruct((NC, N_NODES, NUM_FILTERS), jnp.float32),
    scratch_types=[pltpu.VMEM((CH,), jnp.int32),
                   pltpu.VMEM((CH, NUM_FILTERS), jnp.float32),
                   pltpu.VMEM_SHARED((N_NODES, NUM_FILTERS), jnp.float32)])
def _sc_scatter(msg_hbm, row_hbm, zero_hbm, out_hbm, idx_v, rows_v, acc_sh):
    c = lax.axis_index("c")
    s = lax.axis_index("s")
    wid = s * NC + c
    base = wid * EPW

    pltpu.sync_copy(zero_hbm.at[pl.ds(s * SLAB_OFF, SLAB)],
                    acc_sh.at[pl.ds(s * SLAB_OFF, SLAB)])
    plsc.subcore_barrier()

    def body(i, carry):
        off = base + i * CH
        pltpu.sync_copy(row_hbm.at[pl.ds(off, CH)], idx_v)
        pltpu.sync_copy(msg_hbm.at[pl.ds(off, CH)], rows_v)
        pltpu.sync_copy(rows_v, acc_sh.at[idx_v], add=True)
        return carry

    lax.fori_loop(0, NCH, body, 0)
    plsc.subcore_barrier()
    pltpu.sync_copy(acc_sh.at[pl.ds(s * SLAB_OFF, SLAB)],
                    out_hbm.at[c, pl.ds(s * SLAB_OFF, SLAB)])


def kernel(x, edge_index, edge_weight, edge_attr,
           lin1_w, lin2_w, lin2_b, mlp_w1, mlp_b1, mlp_w2, mlp_b2,
           lin_w, lin_b):
    ei = edge_index.astype(jnp.int32)
    row = ei[0]
    col = ei[1]

    xf = _compute_xf(x, lin1_w)
    g = _sc_gather(xf, col)
    msg = _compute_msg(edge_attr, edge_weight.reshape(N_EDGES, 1), g,
                       mlp_w1, mlp_b1.reshape(1, NUM_FILTERS),
                       mlp_w2, mlp_b2.reshape(1, NUM_FILTERS))
    zeros = jnp.zeros((N_NODES, NUM_FILTERS), jnp.float32)
    parts = _sc_scatter(msg, row, zeros)
    out = _compute_tail(parts, lin2_w, lin2_b.reshape(1, HIDDEN),
                        lin_w, lin_b.reshape(1, HIDDEN))
    return out
